# baseline (speedup 1.0000x reference)
import functools

import jax
import jax.numpy as jnp
from jax import lax
from jax.experimental import pallas as pl
from jax.experimental.pallas import tpu as pltpu
from jax.experimental.pallas import tpu_sc as plsc

B = 16384
EMB = 32
HID = 32
N_USERS = 1000000
N_MOVIES = 100000
NC = 2
NS = 16
NW = NC * NS
BPW = B // NW
CHUNK = 128
NCHUNK = BPW // CHUNK
PK = 128 // EMB
RPW = BPW // PK
L = 16


def _sc_gather(u_idx2d, m_idx2d, ut_w, mt_w):
    mesh = plsc.VectorSubcoreMesh(core_axis_name="c", subcore_axis_name="s")

    @functools.partial(
        pl.kernel,
        mesh=mesh,
        compiler_params=pltpu.CompilerParams(use_tc_tiling_on_sc=False, needs_layout_passes=False),
        out_type=(
            jax.ShapeDtypeStruct((B // PK, 128), jnp.float32),
            jax.ShapeDtypeStruct((B // PK, 128), jnp.float32),
        ),
        scratch_types=[
            pltpu.VMEM((NCHUNK, CHUNK), jnp.int32),
            pltpu.VMEM((NCHUNK, CHUNK), jnp.int32),
            pltpu.VMEM((NCHUNK, CHUNK), jnp.int32),
            pltpu.VMEM((NCHUNK, CHUNK), jnp.int32),
            pltpu.VMEM((CHUNK, 128), jnp.float32),
            pltpu.VMEM((CHUNK, 128), jnp.float32),
            pltpu.VMEM((RPW, 128), jnp.float32),
            pltpu.VMEM((RPW, 128), jnp.float32),
            pltpu.SemaphoreType.DMA,
            pltpu.SemaphoreType.DMA,
        ],
    )
    def k(u_idx_hbm, m_idx_hbm, ut_hbm, mt_hbm, u_out, m_out,
          uidx_v, midx_v, updx_v, mpdx_v, ubuf_v, mbuf_v, upk_v, mpk_v,
          sem_u, sem_m):
        wid = lax.axis_index("s") * NC + lax.axis_index("c")
        pltpu.sync_copy(u_idx_hbm.at[pl.ds(wid * NCHUNK, NCHUNK)], uidx_v)
        pltpu.sync_copy(m_idx_hbm.at[pl.ds(wid * NCHUNK, NCHUNK)], midx_v)
        for j in range(NCHUNK):
            for g in range(CHUNK // L):
                sl = pl.ds(g * L, L)
                updx_v[j, sl] = lax.shift_right_logical(uidx_v[j, sl], 2)
                mpdx_v[j, sl] = lax.shift_right_logical(midx_v[j, sl], 2)

        iota = lax.iota(jnp.int32, L)

        def extract(idx_ref, buf_ref, pk_ref, j):
            def g_body(g, _):
                rows = g * L + iota
                uv = idx_ref[j, pl.ds(g * L, L)]
                seg = lax.bitwise_and(uv, 3) * EMB
                prow = j * (CHUNK // PK) + lax.shift_right_logical(rows, 2)
                pcol0 = lax.bitwise_and(rows, 3) * EMB
                for c in range(EMB):
                    vals = plsc.load_gather(buf_ref, [rows, seg + c])
                    plsc.store_scatter(pk_ref, [prow, pcol0 + c], vals)
                return 0

            lax.fori_loop(0, CHUNK // L, g_body, 0)

        def chunk_body(j, _):
            cu = pltpu.async_copy(ut_hbm.at[updx_v.at[j]], ubuf_v, sem_u)
            cm = pltpu.async_copy(mt_hbm.at[mpdx_v.at[j]], mbuf_v, sem_m)
            cu.wait()
            extract(uidx_v, ubuf_v, upk_v, j)
            cm.wait()
            extract(midx_v, mbuf_v, mpk_v, j)
            return 0

        lax.fori_loop(0, NCHUNK, chunk_body, 0)

        pltpu.sync_copy(upk_v, u_out.at[pl.ds(wid * RPW, RPW)])
        pltpu.sync_copy(mpk_v, m_out.at[pl.ds(wid * RPW, RPW)])

    return k(u_idx2d, m_idx2d, ut_w, mt_w)


def _tc_mlp(u4, m4, w1u_bd, w1m_bd, b1_t, w2_bd, b2_2d):
    BLK = 512

    def body(u_ref, m_ref, w1u_ref, w1m_ref, b1_ref, w2_ref, b2_ref, o_ref):
        h = jnp.dot(u_ref[...], w1u_ref[...], preferred_element_type=jnp.float32)
        h = h + jnp.dot(m_ref[...], w1m_ref[...], preferred_element_type=jnp.float32)
        h = jnp.maximum(h + b1_ref[...], 0.0)
        o = jnp.dot(h, w2_ref[...], preferred_element_type=jnp.float32) + b2_ref[0, 0]
        o_ref[...] = jnp.maximum(o, 0.0)

    return pl.pallas_call(
        body,
        grid=(B // PK // BLK,),
        in_specs=[
            pl.BlockSpec((BLK, 128), lambda i: (i, 0)),
            pl.BlockSpec((BLK, 128), lambda i: (i, 0)),
            pl.BlockSpec((128, 128), lambda i: (0, 0)),
            pl.BlockSpec((128, 128), lambda i: (0, 0)),
            pl.BlockSpec((1, 128), lambda i: (0, 0)),
            pl.BlockSpec((128, PK), lambda i: (0, 0)),
            pl.BlockSpec((1, 1), lambda i: (0, 0)),
        ],
        out_specs=pl.BlockSpec((BLK, PK), lambda i: (i, 0)),
        out_shape=jax.ShapeDtypeStruct((B // PK, PK), jnp.float32),
    )(u4, m4, w1u_bd, w1m_bd, b1_t, w2_bd, b2_2d)


def kernel(u_idx, m_idx, user_table, movie_table, W1, b1, W2, b2):
    u_idx2d = u_idx.astype(jnp.int32).reshape(B // CHUNK, CHUNK)
    m_idx2d = m_idx.astype(jnp.int32).reshape(B // CHUNK, CHUNK)
    ut_w = user_table.reshape(N_USERS // PK, 128)
    mt_w = movie_table.reshape(N_MOVIES // PK, 128)
    u4, m4 = _sc_gather(u_idx2d, m_idx2d, ut_w, mt_w)
    eye = jnp.eye(PK, dtype=jnp.float32)
    w1u_bd = jnp.kron(eye, W1[:, :EMB].T)
    w1m_bd = jnp.kron(eye, W1[:, EMB:].T)
    w2_bd = jnp.kron(eye, W2.T)
    b1_t = jnp.tile(b1, PK).reshape(1, 128)
    out4 = _tc_mlp(u4, m4, w1u_bd, w1m_bd, b1_t, w2_bd, b2.reshape(1, 1))
    return out4.reshape(B)


# speedup vs baseline: 1.0466x; 1.0466x over previous
import functools

import jax
import jax.numpy as jnp
from jax import lax
from jax.experimental import pallas as pl
from jax.experimental.pallas import tpu as pltpu
from jax.experimental.pallas import tpu_sc as plsc

B = 16384
EMB = 32
HID = 32
NC = 2
NS = 16
NW = NC * NS
BPW = B // NW
CHUNK = 128
NCHUNK = BPW // CHUNK


def _sc_gather(u_idx2d, m_idx2d, user_table, movie_table):
    mesh = plsc.VectorSubcoreMesh(core_axis_name="c", subcore_axis_name="s")

    @functools.partial(
        pl.kernel,
        mesh=mesh,
        compiler_params=pltpu.CompilerParams(use_tc_tiling_on_sc=False),
        out_type=(
            jax.ShapeDtypeStruct((B, EMB), jnp.float32),
            jax.ShapeDtypeStruct((B, EMB), jnp.float32),
        ),
        scratch_types=[
            pltpu.VMEM((NCHUNK, CHUNK), jnp.int32),
            pltpu.VMEM((NCHUNK, CHUNK), jnp.int32),
            pltpu.VMEM((BPW, EMB), jnp.float32),
            pltpu.VMEM((BPW, EMB), jnp.float32),
            pltpu.SemaphoreType.DMA,
            pltpu.SemaphoreType.DMA,
        ],
    )
    def k(u_idx_hbm, m_idx_hbm, ut_hbm, mt_hbm, u_out, m_out,
          uidx_v, midx_v, urows_v, mrows_v, sem_u, sem_m):
        wid = lax.axis_index("s") * NC + lax.axis_index("c")
        base = wid * BPW
        pltpu.sync_copy(u_idx_hbm.at[pl.ds(wid * NCHUNK, NCHUNK)], uidx_v)
        pltpu.sync_copy(m_idx_hbm.at[pl.ds(wid * NCHUNK, NCHUNK)], midx_v)
        copies = []
        for j in range(NCHUNK):
            copies.append(pltpu.async_copy(
                ut_hbm.at[uidx_v.at[j]],
                urows_v.at[pl.ds(j * CHUNK, CHUNK)], sem_u))
            copies.append(pltpu.async_copy(
                mt_hbm.at[midx_v.at[j]],
                mrows_v.at[pl.ds(j * CHUNK, CHUNK)], sem_m))
        for c in copies:
            c.wait()
        pltpu.sync_copy(urows_v, u_out.at[pl.ds(base, BPW)])
        pltpu.sync_copy(mrows_v, m_out.at[pl.ds(base, BPW)])

    return k(u_idx2d, m_idx2d, user_table, movie_table)


def _tc_mlp(u_rows, m_rows, w1u_t, w1m_t, b1_2d, w2_2d, b2_2d):
    BLK = 2048

    def body(u_ref, m_ref, w1u_ref, w1m_ref, b1_ref, w2_ref, b2_ref, o_ref):
        h = jnp.dot(u_ref[...], w1u_ref[...], preferred_element_type=jnp.float32)
        h = h + jnp.dot(m_ref[...], w1m_ref[...], preferred_element_type=jnp.float32)
        h = jnp.maximum(h + b1_ref[...], 0.0)
        o = jnp.sum(h * w2_ref[...], axis=1, keepdims=True) + b2_ref[0, 0]
        o_ref[...] = jnp.maximum(o, 0.0)

    out = pl.pallas_call(
        body,
        grid=(B // BLK,),
        in_specs=[
            pl.BlockSpec((BLK, EMB), lambda i: (i, 0)),
            pl.BlockSpec((BLK, EMB), lambda i: (i, 0)),
            pl.BlockSpec((EMB, HID), lambda i: (0, 0)),
            pl.BlockSpec((EMB, HID), lambda i: (0, 0)),
            pl.BlockSpec((1, HID), lambda i: (0, 0)),
            pl.BlockSpec((1, HID), lambda i: (0, 0)),
            pl.BlockSpec((1, 1), lambda i: (0, 0)),
        ],
        out_specs=pl.BlockSpec((BLK, 1), lambda i: (i, 0)),
        out_shape=jax.ShapeDtypeStruct((B, 1), jnp.float32),
    )(u_rows, m_rows, w1u_t, w1m_t, b1_2d, w2_2d, b2_2d)
    return out[:, 0]


def kernel(u_idx, m_idx, user_table, movie_table, W1, b1, W2, b2):
    u_idx2d = u_idx.astype(jnp.int32).reshape(B // CHUNK, CHUNK)
    m_idx2d = m_idx.astype(jnp.int32).reshape(B // CHUNK, CHUNK)
    u_rows, m_rows = _sc_gather(u_idx2d, m_idx2d, user_table, movie_table)
    w1u_t = W1[:, :EMB].T
    w1m_t = W1[:, EMB:].T
    return _tc_mlp(u_rows, m_rows, w1u_t, w1m_t,
                   b1.reshape(1, HID), W2, b2.reshape(1, 1))
